# Initial kernel scaffold; baseline (speedup 1.0000x reference)
import functools

import jax
import jax.numpy as jnp
from jax import lax
from jax.experimental import pallas as pl
from jax.experimental.pallas import tpu as pltpu
from jax.experimental.pallas import tpu_sc as plsc

PITCH_NUM = 128
BEAT_NUM = 64
DUR_NUM = 64
EMB = 128
B = 1024
L = 200

NW = 32
CHUNK = 100
CHUNKS_PER_W = (B * L) // (NW * CHUNK)
BATCH_PER_W = B // NW


@functools.partial(
    pl.kernel,
    out_type=(
        jax.ShapeDtypeStruct((B * 2 * L, EMB), jnp.float32),
        jax.ShapeDtypeStruct((B * L, EMB), jnp.float32),
    ),
    mesh=plsc.VectorSubcoreMesh(core_axis_name="c", subcore_axis_name="s"),
    scratch_types=(
        pltpu.VMEM((CHUNKS_PER_W, CHUNK), jnp.int32),
        pltpu.VMEM((CHUNKS_PER_W, CHUNK), jnp.int32),
        pltpu.VMEM((CHUNKS_PER_W, CHUNK), jnp.int32),
        pltpu.VMEM((CHUNK, EMB), jnp.float32),
        pltpu.VMEM((CHUNK, EMB), jnp.float32),
        pltpu.VMEM((CHUNK, EMB), jnp.float32),
        pltpu.SemaphoreType.DMA,
        pltpu.SemaphoreType.DMA,
        pltpu.SemaphoreType.DMA,
    ),
)
def _sc_lookup(pitch_hbm, dur_hbm, beat_hbm,
               pitch_emb, dur_emb, beat_emb,
               out_cat, out_be,
               pidx_v, didx_v, bidx_v,
               pe_buf, de_buf, be_buf,
               sem_p, sem_d, sem_b):
    wid = lax.axis_index("s") * 2 + lax.axis_index("c")
    row0 = wid * CHUNKS_PER_W

    pltpu.sync_copy(pitch_hbm.at[pl.ds(row0, CHUNKS_PER_W)], pidx_v)
    pltpu.sync_copy(dur_hbm.at[pl.ds(row0, CHUNKS_PER_W)], didx_v)
    pltpu.sync_copy(beat_hbm.at[pl.ds(row0, CHUNKS_PER_W)], bidx_v)

    def body(j, carry):
        b = wid * BATCH_PER_W + j // 2
        h = j % 2
        tok = (row0 + j) * CHUNK
        pe_dest = b * (2 * L) + h * CHUNK
        de_dest = b * (2 * L) + L + h * CHUNK

        cp = pltpu.async_copy(pitch_emb.at[pidx_v.at[j]], pe_buf, sem_p)
        cd = pltpu.async_copy(dur_emb.at[didx_v.at[j]], de_buf, sem_d)
        cb = pltpu.async_copy(beat_emb.at[bidx_v.at[j]], be_buf, sem_b)
        cp.wait()
        pltpu.sync_copy(pe_buf, out_cat.at[pl.ds(pe_dest, CHUNK)])
        cd.wait()
        pltpu.sync_copy(de_buf, out_cat.at[pl.ds(de_dest, CHUNK)])
        cb.wait()
        pltpu.sync_copy(be_buf, out_be.at[pl.ds(tok, CHUNK)])
        return carry

    lax.fori_loop(0, CHUNKS_PER_W, body, 0)


def kernel(x, beat_info, pitch_emb, beat_emb, dur_emb):
    pitch = x[..., 2]
    dur = x[..., 3]
    beat = beat_info

    n_rows = (B * L) // CHUNK
    pitch2d = pitch.reshape(n_rows, CHUNK)
    dur2d = dur.reshape(n_rows, CHUNK)
    beat2d = beat.reshape(n_rows, CHUNK)

    out_cat_rows, be_rows = _sc_lookup(pitch2d, dur2d, beat2d,
                                       pitch_emb, dur_emb, beat_emb)
    out_cat = out_cat_rows.reshape(B, 2 * L, EMB)
    be = be_rows.reshape(B, L, EMB)
    return (out_cat, be, beat, pitch, dur)


# speedup vs baseline: 9.7987x; 9.7987x over previous
import functools

import jax
import jax.numpy as jnp
from jax import lax
from jax.experimental import pallas as pl
from jax.experimental.pallas import tpu as pltpu
from jax.experimental.pallas import tpu_sc as plsc

PITCH_NUM = 128
BEAT_NUM = 64
DUR_NUM = 64
EMB = 128
B = 1024
L = 200

NW = 32
CHUNK = 40
CPB = L // CHUNK
CHUNKS_PER_W = (B * L) // (NW * CHUNK)
BATCH_PER_W = B // NW


@functools.partial(
    pl.kernel,
    out_type=(
        jax.ShapeDtypeStruct((B * 2 * L, EMB), jnp.float32),
        jax.ShapeDtypeStruct((B * L, EMB), jnp.float32),
    ),
    mesh=plsc.VectorSubcoreMesh(core_axis_name="c", subcore_axis_name="s"),
    scratch_types=(
        pltpu.VMEM_SHARED((PITCH_NUM, EMB), jnp.float32),
        pltpu.VMEM_SHARED((DUR_NUM, EMB), jnp.float32),
        pltpu.VMEM_SHARED((BEAT_NUM, EMB), jnp.float32),
        pltpu.VMEM((CHUNKS_PER_W, CHUNK), jnp.int32),
        pltpu.VMEM((CHUNKS_PER_W, CHUNK), jnp.int32),
        pltpu.VMEM((CHUNKS_PER_W, CHUNK), jnp.int32),
        pltpu.VMEM((CHUNK, EMB), jnp.float32),
        pltpu.VMEM((CHUNK, EMB), jnp.float32),
        pltpu.VMEM((CHUNK, EMB), jnp.float32),
        pltpu.SemaphoreType.DMA,
        pltpu.SemaphoreType.DMA,
        pltpu.SemaphoreType.DMA,
    ),
)
def _sc_lookup(pitch_hbm, dur_hbm, beat_hbm,
               pitch_emb, dur_emb, beat_emb,
               out_cat, out_be,
               pitch_sh, dur_sh, beat_sh,
               pidx_v, didx_v, bidx_v,
               pe_buf, de_buf, be_buf,
               sem_p, sem_d, sem_b):
    sid = lax.axis_index("s")
    wid = sid * 2 + lax.axis_index("c")
    row0 = wid * CHUNKS_PER_W

    @pl.when(sid == 0)
    def _stage_tables():
        pltpu.sync_copy(pitch_emb, pitch_sh)
        pltpu.sync_copy(dur_emb, dur_sh)
        pltpu.sync_copy(beat_emb, beat_sh)

    pltpu.sync_copy(pitch_hbm.at[pl.ds(row0, CHUNKS_PER_W)], pidx_v)
    pltpu.sync_copy(dur_hbm.at[pl.ds(row0, CHUNKS_PER_W)], didx_v)
    pltpu.sync_copy(beat_hbm.at[pl.ds(row0, CHUNKS_PER_W)], bidx_v)

    plsc.subcore_barrier()

    def body(j, carry):
        b = wid * BATCH_PER_W + j // CPB
        h = j % CPB
        tok = (row0 + j) * CHUNK
        pe_dest = b * (2 * L) + h * CHUNK
        de_dest = b * (2 * L) + L + h * CHUNK

        cp = pltpu.async_copy(pitch_sh.at[pidx_v.at[j]], pe_buf, sem_p)
        cd = pltpu.async_copy(dur_sh.at[didx_v.at[j]], de_buf, sem_d)
        cb = pltpu.async_copy(beat_sh.at[bidx_v.at[j]], be_buf, sem_b)
        cp.wait()
        pltpu.sync_copy(pe_buf, out_cat.at[pl.ds(pe_dest, CHUNK)])
        cd.wait()
        pltpu.sync_copy(de_buf, out_cat.at[pl.ds(de_dest, CHUNK)])
        cb.wait()
        pltpu.sync_copy(be_buf, out_be.at[pl.ds(tok, CHUNK)])
        return carry

    lax.fori_loop(0, CHUNKS_PER_W, body, 0)


def kernel(x, beat_info, pitch_emb, beat_emb, dur_emb):
    pitch = x[..., 2]
    dur = x[..., 3]
    beat = beat_info

    n_rows = (B * L) // CHUNK
    pitch2d = pitch.reshape(n_rows, CHUNK)
    dur2d = dur.reshape(n_rows, CHUNK)
    beat2d = beat.reshape(n_rows, CHUNK)

    out_cat_rows, be_rows = _sc_lookup(pitch2d, dur2d, beat2d,
                                       pitch_emb, dur_emb, beat_emb)
    out_cat = out_cat_rows.reshape(B, 2 * L, EMB)
    be = be_rows.reshape(B, L, EMB)
    return (out_cat, be, beat, pitch, dur)


# speedup vs baseline: 13.7333x; 1.4015x over previous
import functools

import jax
import jax.numpy as jnp
from jax import lax
from jax.experimental import pallas as pl
from jax.experimental.pallas import tpu as pltpu
from jax.experimental.pallas import tpu_sc as plsc

PITCH_NUM = 128
BEAT_NUM = 64
DUR_NUM = 64
EMB = 128
B = 1024
L = 200

NW = 32
CHUNK = 40
CPB = L // CHUNK
NCH = (B * L) // (NW * CHUNK)
BATCH_PER_W = B // NW
NBUF = 2


@functools.partial(
    pl.kernel,
    out_type=(
        jax.ShapeDtypeStruct((B * 2 * L, EMB), jnp.float32),
        jax.ShapeDtypeStruct((B * L, EMB), jnp.float32),
    ),
    mesh=plsc.VectorSubcoreMesh(core_axis_name="c", subcore_axis_name="s"),
    scratch_types=(
        pltpu.VMEM_SHARED((PITCH_NUM, EMB), jnp.float32),
        pltpu.VMEM_SHARED((DUR_NUM, EMB), jnp.float32),
        pltpu.VMEM_SHARED((BEAT_NUM, EMB), jnp.float32),
        pltpu.VMEM((NCH, CHUNK), jnp.int32),
        pltpu.VMEM((NCH, CHUNK), jnp.int32),
        pltpu.VMEM((NCH, CHUNK), jnp.int32),
        pltpu.VMEM((CHUNK, EMB), jnp.float32),
        pltpu.VMEM((CHUNK, EMB), jnp.float32),
        pltpu.VMEM((CHUNK, EMB), jnp.float32),
        pltpu.VMEM((CHUNK, EMB), jnp.float32),
        pltpu.VMEM((CHUNK, EMB), jnp.float32),
        pltpu.VMEM((CHUNK, EMB), jnp.float32),
        pltpu.SemaphoreType.DMA,
        pltpu.SemaphoreType.DMA,
        pltpu.SemaphoreType.DMA,
        pltpu.SemaphoreType.DMA,
    ),
)
def _sc_lookup(pitch_hbm, dur_hbm, beat_hbm,
               pitch_emb, dur_emb, beat_emb,
               out_cat, out_be,
               pitch_sh, dur_sh, beat_sh,
               pidx_v, didx_v, bidx_v,
               pe_buf0, pe_buf1, de_buf0, de_buf1, be_buf0, be_buf1,
               gsem0, gsem1, wsem0, wsem1):
    sid = lax.axis_index("s")
    wid = sid * 2 + lax.axis_index("c")
    row0 = wid * NCH

    pe_buf = (pe_buf0, pe_buf1)
    de_buf = (de_buf0, de_buf1)
    be_buf = (be_buf0, be_buf1)
    gsem = (gsem0, gsem1)
    wsem = (wsem0, wsem1)

    @pl.when(sid == 0)
    def _stage_tables():
        pltpu.sync_copy(pitch_emb, pitch_sh)
        pltpu.sync_copy(dur_emb, dur_sh)
        pltpu.sync_copy(beat_emb, beat_sh)

    pltpu.sync_copy(pitch_hbm.at[pl.ds(row0, NCH)], pidx_v)
    pltpu.sync_copy(dur_hbm.at[pl.ds(row0, NCH)], didx_v)
    pltpu.sync_copy(beat_hbm.at[pl.ds(row0, NCH)], bidx_v)

    plsc.subcore_barrier()

    def gather_descs(j, s):
        return (
            pltpu.make_async_copy(pitch_sh.at[pidx_v.at[j]], pe_buf[s], gsem[s]),
            pltpu.make_async_copy(dur_sh.at[didx_v.at[j]], de_buf[s], gsem[s]),
            pltpu.make_async_copy(beat_sh.at[bidx_v.at[j]], be_buf[s], gsem[s]),
        )

    def write_descs(j, s):
        b = wid * BATCH_PER_W + j // CPB
        h = j % CPB
        pe_dest = b * (2 * L) + h * CHUNK
        de_dest = b * (2 * L) + L + h * CHUNK
        tok = (row0 + j) * CHUNK
        return (
            pltpu.make_async_copy(pe_buf[s], out_cat.at[pl.ds(pe_dest, CHUNK)], wsem[s]),
            pltpu.make_async_copy(de_buf[s], out_cat.at[pl.ds(de_dest, CHUNK)], wsem[s]),
            pltpu.make_async_copy(be_buf[s], out_be.at[pl.ds(tok, CHUNK)], wsem[s]),
        )

    def issue(descs):
        for d in descs:
            d.start()

    def drain(descs):
        for d in descs:
            d.wait()

    issue(gather_descs(0, 0))

    def outer(g, carry):
        for s in range(NBUF):
            j = g * NBUF + s
            drain(gather_descs(j, s))
            issue(write_descs(j, s))
            sp = (s + NBUF - 1) % NBUF
            jn = j + NBUF - 1

            @pl.when(j >= 1)
            def _drain_prev_writes():
                drain(write_descs(j - 1, sp))

            @pl.when(jn < NCH)
            def _issue_next_gather():
                issue(gather_descs(jn, sp))
        return carry

    lax.fori_loop(0, NCH // NBUF, outer, 0)
    drain(write_descs(NCH - 1, (NCH - 1) % NBUF))


def kernel(x, beat_info, pitch_emb, beat_emb, dur_emb):
    pitch = x[..., 2]
    dur = x[..., 3]
    beat = beat_info

    n_rows = (B * L) // CHUNK
    pitch2d = pitch.reshape(n_rows, CHUNK)
    dur2d = dur.reshape(n_rows, CHUNK)
    beat2d = beat.reshape(n_rows, CHUNK)

    out_cat_rows, be_rows = _sc_lookup(pitch2d, dur2d, beat2d,
                                       pitch_emb, dur_emb, beat_emb)
    out_cat = out_cat_rows.reshape(B, 2 * L, EMB)
    be = be_rows.reshape(B, L, EMB)
    return (out_cat, be, beat, pitch, dur)


# speedup vs baseline: 14.8454x; 1.0810x over previous
import functools

import jax
import jax.numpy as jnp
from jax import lax
from jax.experimental import pallas as pl
from jax.experimental.pallas import tpu as pltpu
from jax.experimental.pallas import tpu_sc as plsc

PITCH_NUM = 128
BEAT_NUM = 64
DUR_NUM = 64
EMB = 128
B = 1024
L = 200

NW = 32
NFIELD = 3
NITEMS = NFIELD * B
IPW = NITEMS // NW
NBUF = 3
SPLIT = 128
TBL = PITCH_NUM + DUR_NUM + BEAT_NUM


@functools.partial(
    pl.kernel,
    out_type=(
        jax.ShapeDtypeStruct((B * 2 * L, EMB), jnp.float32),
        jax.ShapeDtypeStruct((B * L, EMB), jnp.float32),
    ),
    mesh=plsc.VectorSubcoreMesh(core_axis_name="c", subcore_axis_name="s"),
    scratch_types=(
        pltpu.VMEM_SHARED((TBL, EMB), jnp.float32),
        pltpu.VMEM((IPW, SPLIT), jnp.int32),
        pltpu.VMEM((IPW, L - SPLIT), jnp.int32),
        pltpu.VMEM((L, EMB), jnp.float32),
        pltpu.VMEM((L, EMB), jnp.float32),
        pltpu.VMEM((L, EMB), jnp.float32),
        pltpu.SemaphoreType.DMA,
        pltpu.SemaphoreType.DMA,
        pltpu.SemaphoreType.DMA,
        pltpu.SemaphoreType.DMA,
        pltpu.SemaphoreType.DMA,
        pltpu.SemaphoreType.DMA,
    ),
)
def _sc_lookup(idxa_hbm, idxb_hbm, table_hbm,
               out_cat, out_be,
               table_sh,
               idxa_v, idxb_v,
               buf0, buf1, buf2,
               gsem0, gsem1, gsem2, wsem0, wsem1, wsem2):
    sid = lax.axis_index("s")
    wid = sid * 2 + lax.axis_index("c")
    q0 = wid * IPW

    buf = (buf0, buf1, buf2)
    gsem = (gsem0, gsem1, gsem2)
    wsem = (wsem0, wsem1, wsem2)

    @pl.when(sid == 0)
    def _stage_table():
        pltpu.sync_copy(table_hbm, table_sh)

    pltpu.sync_copy(idxa_hbm.at[pl.ds(q0, IPW)], idxa_v)
    pltpu.sync_copy(idxb_hbm.at[pl.ds(q0, IPW)], idxb_v)

    plsc.subcore_barrier()

    def gather_descs(i, s):
        return (
            pltpu.make_async_copy(table_sh.at[idxa_v.at[i]],
                                  buf[s].at[pl.ds(0, SPLIT)], gsem[s]),
            pltpu.make_async_copy(table_sh.at[idxb_v.at[i]],
                                  buf[s].at[pl.ds(SPLIT, L - SPLIT)], gsem[s]),
        )

    def issue_gathers(i, s):
        for d in gather_descs(i, s):
            d.start()

    def drain_gathers(i, s):
        for d in gather_descs(i, s):
            d.wait()

    def issue_write(i, s):
        q = q0 + i
        f = q // B
        b = q - f * B

        @pl.when(f < 2)
        def _to_cat():
            pltpu.make_async_copy(
                buf[s], out_cat.at[pl.ds(b * (2 * L) + f * L, L)], wsem[s]
            ).start()

        @pl.when(f >= 2)
        def _to_be():
            pltpu.make_async_copy(
                buf[s], out_be.at[pl.ds(b * L, L)], wsem[s]
            ).start()

    def drain_write(s):
        pltpu.make_async_copy(buf[s], out_cat.at[pl.ds(0, L)], wsem[s]).wait()

    issue_gathers(0, 0)
    issue_gathers(1, 1)

    def outer(g, carry):
        for s in range(NBUF):
            i = g * NBUF + s
            drain_gathers(i, s)
            issue_write(i, s)
            sp = (s + 2) % NBUF
            nxt = i + 2

            @pl.when(i >= 1)
            def _drain_prev_write():
                drain_write(sp)

            @pl.when(nxt < IPW)
            def _issue_next_gather():
                issue_gathers(nxt, sp)
        return carry

    lax.fori_loop(0, IPW // NBUF, outer, 0)
    drain_write((IPW - 1) % NBUF)


def kernel(x, beat_info, pitch_emb, beat_emb, dur_emb):
    pitch = x[..., 2]
    dur = x[..., 3]
    beat = beat_info

    table = jnp.concatenate([pitch_emb, dur_emb, beat_emb], axis=0)
    idx = jnp.concatenate(
        [pitch, dur + PITCH_NUM, beat + (PITCH_NUM + DUR_NUM)], axis=0
    )
    idxa = idx[:, :SPLIT]
    idxb = idx[:, SPLIT:]

    out_cat_rows, be_rows = _sc_lookup(idxa, idxb, table)
    out_cat = out_cat_rows.reshape(B, 2 * L, EMB)
    be = be_rows.reshape(B, L, EMB)
    return (out_cat, be, beat, pitch, dur)
